# baseline (device time: 1265156 ns/iter reference)
import jax
import jax.numpy as jnp
from jax import lax
from jax.experimental import pallas as pl
from jax.experimental.pallas import tpu as pltpu

N_DEV = 4
M_PER = 2048
HALF = M_PER // 2
K = 8192
N_PER = 1024


def _gather_body(x_ref, top_ref, bot_ref, sa_send, sa_recv, sb_send, sb_recv,
                 cp_sems):
    my = lax.axis_index("i")
    right = lax.rem(my + 1, N_DEV)
    left = lax.rem(my + N_DEV - 1, N_DEV)

    barrier = pltpu.get_barrier_semaphore()
    for nbr in (left, right):
        pl.semaphore_signal(
            barrier, inc=1, device_id=(nbr,),
            device_id_type=pl.DeviceIdType.MESH,
        )
    pl.semaphore_wait(barrier, 2)

    cp_t = pltpu.make_async_copy(
        x_ref.at[pl.ds(0, HALF), :],
        top_ref.at[pl.ds(my * HALF, HALF), :], cp_sems.at[0],
    )
    cp_b = pltpu.make_async_copy(
        x_ref.at[pl.ds(HALF, HALF), :],
        bot_ref.at[pl.ds(my * HALF, HALF), :], cp_sems.at[1],
    )
    cp_t.start()
    cp_b.start()

    for h in range(N_DEV - 1):
        oa = lax.rem(my - h + N_DEV, N_DEV)
        ob = lax.rem(my + h, N_DEV)
        if h == 0:
            src_a = x_ref.at[pl.ds(0, HALF), :]
            src_b = x_ref.at[pl.ds(HALF, HALF), :]
        else:
            src_a = top_ref.at[pl.ds(oa * HALF, HALF), :]
            src_b = bot_ref.at[pl.ds(ob * HALF, HALF), :]
        rdma_a = pltpu.make_async_remote_copy(
            src_ref=src_a,
            dst_ref=top_ref.at[pl.ds(oa * HALF, HALF), :],
            send_sem=sa_send.at[h],
            recv_sem=sa_recv.at[h],
            device_id=(right,),
            device_id_type=pl.DeviceIdType.MESH,
        )
        rdma_b = pltpu.make_async_remote_copy(
            src_ref=src_b,
            dst_ref=bot_ref.at[pl.ds(ob * HALF, HALF), :],
            send_sem=sb_send.at[h],
            recv_sem=sb_recv.at[h],
            device_id=(left,),
            device_id_type=pl.DeviceIdType.MESH,
        )
        rdma_a.start()
        rdma_b.start()
        rdma_a.wait()
        rdma_b.wait()

    cp_t.wait()
    cp_b.wait()


def _all_gather(x_bf16):
    return pl.pallas_call(
        _gather_body,
        out_shape=[
            jax.ShapeDtypeStruct((N_DEV * HALF, K), jnp.bfloat16),
            jax.ShapeDtypeStruct((N_DEV * HALF, K), jnp.bfloat16),
        ],
        in_specs=[pl.BlockSpec(memory_space=pl.ANY)],
        out_specs=[
            pl.BlockSpec(memory_space=pl.ANY),
            pl.BlockSpec(memory_space=pl.ANY),
        ],
        scratch_shapes=[
            pltpu.SemaphoreType.DMA((N_DEV - 1,)),
            pltpu.SemaphoreType.DMA((N_DEV - 1,)),
            pltpu.SemaphoreType.DMA((N_DEV - 1,)),
            pltpu.SemaphoreType.DMA((N_DEV - 1,)),
            pltpu.SemaphoreType.DMA((2,)),
        ],
        compiler_params=pltpu.CompilerParams(collective_id=0),
    )(x_bf16)


def _gemm_body(xg_ref, w_ref, o_ref):
    y = jnp.dot(xg_ref[...], w_ref[...], preferred_element_type=jnp.float32)
    o_ref[...] = y * jax.nn.sigmoid(y)


def _gemm_silu(xg, w_bf16):
    m, _ = xg.shape
    m_blk = 512
    return pl.pallas_call(
        _gemm_body,
        grid=(m // m_blk,),
        in_specs=[
            pl.BlockSpec((m_blk, K), lambda i: (i, 0)),
            pl.BlockSpec((K, N_PER), lambda i: (0, 0)),
        ],
        out_specs=pl.BlockSpec((m_blk, N_PER), lambda i: (i, 0)),
        out_shape=jax.ShapeDtypeStruct((m, N_PER), jnp.float32),
        compiler_params=pltpu.CompilerParams(
            dimension_semantics=("arbitrary",),
            vmem_limit_bytes=60 * 1024 * 1024,
        ),
    )(xg, w_bf16)


def kernel(x, w_mat):
    x_bf16 = x.astype(jnp.bfloat16)
    w_bf16 = w_mat.astype(jnp.bfloat16)
    top, bot = _all_gather(x_bf16)
    out_top = _gemm_silu(top, w_bf16)
    out_bot = _gemm_silu(bot, w_bf16)
    ot = out_top.reshape(N_DEV, HALF, N_PER)
    ob = out_bot.reshape(N_DEV, HALF, N_PER)
    return jnp.concatenate([ot, ob], axis=1).reshape(N_DEV * M_PER, N_PER)


# device time: 686554 ns/iter; 1.8428x vs baseline; 1.8428x over previous
import jax
import jax.numpy as jnp
from jax import lax
from jax.experimental import pallas as pl
from jax.experimental.pallas import tpu as pltpu

N_DEV = 4
M_PER = 2048
HALF = M_PER // 2
TILE = 512
K = 8192
N_PER = 1024
N_TILES = N_DEV * M_PER // TILE


def _fused_body(x_ref, w_ref, out_ref, top_ref, bot_ref,
                sa_send, sa_recv, sb_send, sb_recv,
                xtile, ostage, ld_sem, st_sem):
    my = lax.axis_index("i")
    right = lax.rem(my + 1, N_DEV)
    left = lax.rem(my + N_DEV - 1, N_DEV)

    barrier = pltpu.get_barrier_semaphore()
    for nbr in (left, right):
        pl.semaphore_signal(
            barrier, inc=1, device_id=(nbr,),
            device_id_type=pl.DeviceIdType.MESH,
        )
    pl.semaphore_wait(barrier, 2)

    def hop_descs(h):
        src_off = jnp.maximum(h - 1, 0) * HALF
        src_a = top_ref.at[pl.ds(src_off, HALF), :]
        src_b = bot_ref.at[pl.ds(src_off, HALF), :]
        ra = pltpu.make_async_remote_copy(
            src_ref=src_a,
            dst_ref=top_ref.at[pl.ds(h * HALF, HALF), :],
            send_sem=sa_send.at[h], recv_sem=sa_recv.at[h],
            device_id=(right,), device_id_type=pl.DeviceIdType.MESH,
        )
        rb = pltpu.make_async_remote_copy(
            src_ref=src_b,
            dst_ref=bot_ref.at[pl.ds(h * HALF, HALF), :],
            send_sem=sb_send.at[h], recv_sem=sb_recv.at[h],
            device_id=(left,), device_id_type=pl.DeviceIdType.MESH,
        )
        return ra, rb

    ra0 = pltpu.make_async_remote_copy(
        src_ref=x_ref.at[pl.ds(0, HALF), :],
        dst_ref=top_ref.at[pl.ds(0, HALF), :],
        send_sem=sa_send.at[0], recv_sem=sa_recv.at[0],
        device_id=(right,), device_id_type=pl.DeviceIdType.MESH,
    )
    rb0 = pltpu.make_async_remote_copy(
        src_ref=x_ref.at[pl.ds(HALF, HALF), :],
        dst_ref=bot_ref.at[pl.ds(0, HALF), :],
        send_sem=sb_send.at[0], recv_sem=sb_recv.at[0],
        device_id=(left,), device_id_type=pl.DeviceIdType.MESH,
    )
    ra0.start()
    rb0.start()

    def tile_body(j, carry):
        at_boundary = jnp.logical_and(j >= 4, lax.rem(j, 4) == 0)
        h_wait = jnp.maximum(j // 4 - 1, 0)

        @pl.when(at_boundary)
        def _():
            ra, rb = hop_descs(h_wait)
            ra.wait()
            rb.wait()

            @pl.when(h_wait + 1 <= N_DEV - 2)
            def _():
                ra2, rb2 = hop_descs(h_wait + 1)
                ra2.start()
                rb2.start()

        is_own = j < 4
        within = lax.rem(j - 4, 4)
        is_top = within < 2
        jj = lax.rem(j, 2)
        own_off = j * TILE
        ring_off = h_wait * HALF + jj * TILE

        @pl.when(is_own)
        def _():
            pltpu.make_async_copy(
                x_ref.at[pl.ds(own_off, TILE), :], xtile, ld_sem
            ).start()

        @pl.when(jnp.logical_and(jnp.logical_not(is_own), is_top))
        def _():
            pltpu.make_async_copy(
                top_ref.at[pl.ds(ring_off, TILE), :], xtile, ld_sem
            ).start()

        @pl.when(jnp.logical_and(jnp.logical_not(is_own),
                                 jnp.logical_not(is_top)))
        def _():
            pltpu.make_async_copy(
                bot_ref.at[pl.ds(ring_off, TILE), :], xtile, ld_sem
            ).start()

        pltpu.make_async_copy(
            x_ref.at[pl.ds(0, TILE), :], xtile, ld_sem
        ).wait()

        y = jnp.dot(xtile[...], w_ref[...],
                    preferred_element_type=jnp.float32)
        y = y * jax.nn.sigmoid(y)

        @pl.when(j >= 1)
        def _():
            pltpu.make_async_copy(
                ostage, out_ref.at[pl.ds(0, TILE), :], st_sem
            ).wait()

        ostage[...] = y

        oa = lax.rem(my - h_wait - 1 + N_DEV, N_DEV)
        ob = lax.rem(my + h_wait + 1, N_DEV)
        out_row = jnp.where(
            is_own,
            my * M_PER + own_off,
            jnp.where(is_top,
                      oa * M_PER + jj * TILE,
                      ob * M_PER + HALF + jj * TILE),
        )
        pltpu.make_async_copy(
            ostage, out_ref.at[pl.ds(out_row, TILE), :], st_sem
        ).start()
        return carry

    lax.fori_loop(0, N_TILES, tile_body, 0)

    pltpu.make_async_copy(
        ostage, out_ref.at[pl.ds(0, TILE), :], st_sem
    ).wait()


def _fused(x_bf16, w_bf16):
    out, _top, _bot = pl.pallas_call(
        _fused_body,
        out_shape=[
            jax.ShapeDtypeStruct((N_DEV * M_PER, N_PER), jnp.float32),
            jax.ShapeDtypeStruct(((N_DEV - 1) * HALF, K), jnp.bfloat16),
            jax.ShapeDtypeStruct(((N_DEV - 1) * HALF, K), jnp.bfloat16),
        ],
        in_specs=[
            pl.BlockSpec(memory_space=pl.ANY),
            pl.BlockSpec(memory_space=pltpu.VMEM),
        ],
        out_specs=[
            pl.BlockSpec(memory_space=pl.ANY),
            pl.BlockSpec(memory_space=pl.ANY),
            pl.BlockSpec(memory_space=pl.ANY),
        ],
        scratch_shapes=[
            pltpu.SemaphoreType.DMA((N_DEV - 1,)),
            pltpu.SemaphoreType.DMA((N_DEV - 1,)),
            pltpu.SemaphoreType.DMA((N_DEV - 1,)),
            pltpu.SemaphoreType.DMA((N_DEV - 1,)),
            pltpu.VMEM((TILE, K), jnp.bfloat16),
            pltpu.VMEM((TILE, N_PER), jnp.float32),
            pltpu.SemaphoreType.DMA,
            pltpu.SemaphoreType.DMA,
        ],
        compiler_params=pltpu.CompilerParams(
            collective_id=0,
            vmem_limit_bytes=60 * 1024 * 1024,
        ),
    )(x_bf16, w_bf16)
    return out


def kernel(x, w_mat):
    x_bf16 = x.astype(jnp.bfloat16)
    w_bf16 = w_mat.astype(jnp.bfloat16)
    return _fused(x_bf16, w_bf16)


# device time: 653563 ns/iter; 1.9358x vs baseline; 1.0505x over previous
import jax
import jax.numpy as jnp
from jax import lax
from jax.experimental import pallas as pl
from jax.experimental.pallas import tpu as pltpu

N_DEV = 4
M_PER = 2048
HALF = M_PER // 2
TILE = 512
N_MSG = 2 * (N_DEV - 1)
K = 8192
N_PER = 1024
N_TILES = N_DEV * M_PER // TILE


def _fused_body(x_ref, w_ref, out_ref, top_ref, bot_ref,
                sa_send, sa_recv, sb_send, sb_recv,
                xtile, ostage, ld_sem, st_sem):
    my = lax.axis_index("i")
    right = lax.rem(my + 1, N_DEV)
    left = lax.rem(my + N_DEV - 1, N_DEV)

    barrier = pltpu.get_barrier_semaphore()
    for nbr in (left, right):
        pl.semaphore_signal(
            barrier, inc=1, device_id=(nbr,),
            device_id_type=pl.DeviceIdType.MESH,
        )
    pl.semaphore_wait(barrier, 2)

    def msg(ring_a, m, src_own):
        if src_own:
            base = 0 if ring_a else HALF
            src = x_ref.at[pl.ds(base + m * TILE, TILE), :]
        else:
            buf = top_ref if ring_a else bot_ref
            off = jnp.maximum(m - 2, 0) * TILE
            src = buf.at[pl.ds(off, TILE), :]
        dst_buf = top_ref if ring_a else bot_ref
        sems = (sa_send, sa_recv) if ring_a else (sb_send, sb_recv)
        tgt = right if ring_a else left
        return pltpu.make_async_remote_copy(
            src_ref=src,
            dst_ref=dst_buf.at[pl.ds(m * TILE, TILE), :],
            send_sem=sems[0].at[m], recv_sem=sems[1].at[m],
            device_id=(tgt,), device_id_type=pl.DeviceIdType.MESH,
        )

    for m in range(2):
        msg(True, m, src_own=True).start()
        msg(False, m, src_own=True).start()

    def tile_body(j, carry):
        is_own = j < 4
        ring_a = lax.rem(j, 2) == 0
        m = jnp.maximum((j - 4) // 2, 0)

        @pl.when(jnp.logical_and(jnp.logical_not(is_own), ring_a))
        def _():
            msg(True, m, src_own=False).wait()

            @pl.when(m < N_MSG - 2)
            def _():
                msg(True, m + 2, src_own=False).start()

            pltpu.make_async_copy(
                top_ref.at[pl.ds(m * TILE, TILE), :], xtile, ld_sem
            ).start()

        @pl.when(jnp.logical_and(jnp.logical_not(is_own),
                                 jnp.logical_not(ring_a)))
        def _():
            msg(False, m, src_own=False).wait()

            @pl.when(m < N_MSG - 2)
            def _():
                msg(False, m + 2, src_own=False).start()

            pltpu.make_async_copy(
                bot_ref.at[pl.ds(m * TILE, TILE), :], xtile, ld_sem
            ).start()

        @pl.when(is_own)
        def _():
            pltpu.make_async_copy(
                x_ref.at[pl.ds(j * TILE, TILE), :], xtile, ld_sem
            ).start()

        pltpu.make_async_copy(
            x_ref.at[pl.ds(0, TILE), :], xtile, ld_sem
        ).wait()

        y = jnp.dot(xtile[...], w_ref[...],
                    preferred_element_type=jnp.float32)
        y = y * jax.nn.sigmoid(y)

        @pl.when(j >= 1)
        def _():
            pltpu.make_async_copy(
                ostage, out_ref.at[pl.ds(0, TILE), :], st_sem
            ).wait()

        ostage[...] = y

        sub = lax.rem(m, 2)
        oa = lax.rem(my - 1 - m // 2 + N_DEV, N_DEV)
        ob = lax.rem(my + 1 + m // 2, N_DEV)
        out_row = jnp.where(
            is_own,
            my * M_PER + j * TILE,
            jnp.where(ring_a,
                      oa * M_PER + sub * TILE,
                      ob * M_PER + HALF + sub * TILE),
        )
        pltpu.make_async_copy(
            ostage, out_ref.at[pl.ds(out_row, TILE), :], st_sem
        ).start()
        return carry

    lax.fori_loop(0, N_TILES, tile_body, 0)

    pltpu.make_async_copy(
        ostage, out_ref.at[pl.ds(0, TILE), :], st_sem
    ).wait()


def _fused(x_bf16, w_bf16):
    out, _top, _bot = pl.pallas_call(
        _fused_body,
        out_shape=[
            jax.ShapeDtypeStruct((N_DEV * M_PER, N_PER), jnp.float32),
            jax.ShapeDtypeStruct((N_MSG * TILE, K), jnp.bfloat16),
            jax.ShapeDtypeStruct((N_MSG * TILE, K), jnp.bfloat16),
        ],
        in_specs=[
            pl.BlockSpec(memory_space=pl.ANY),
            pl.BlockSpec(memory_space=pltpu.VMEM),
        ],
        out_specs=[
            pl.BlockSpec(memory_space=pl.ANY),
            pl.BlockSpec(memory_space=pl.ANY),
            pl.BlockSpec(memory_space=pl.ANY),
        ],
        scratch_shapes=[
            pltpu.SemaphoreType.DMA((N_MSG,)),
            pltpu.SemaphoreType.DMA((N_MSG,)),
            pltpu.SemaphoreType.DMA((N_MSG,)),
            pltpu.SemaphoreType.DMA((N_MSG,)),
            pltpu.VMEM((TILE, K), jnp.bfloat16),
            pltpu.VMEM((TILE, N_PER), jnp.float32),
            pltpu.SemaphoreType.DMA,
            pltpu.SemaphoreType.DMA,
        ],
        compiler_params=pltpu.CompilerParams(
            collective_id=0,
            vmem_limit_bytes=60 * 1024 * 1024,
        ),
    )(x_bf16, w_bf16)
    return out


def kernel(x, w_mat):
    x_bf16 = x.astype(jnp.bfloat16)
    w_bf16 = w_mat.astype(jnp.bfloat16)
    return _fused(x_bf16, w_bf16)


# device time: 629863 ns/iter; 2.0086x vs baseline; 1.0376x over previous
import jax
import jax.numpy as jnp
from jax import lax
from jax.experimental import pallas as pl
from jax.experimental.pallas import tpu as pltpu

N_DEV = 4
M_PER = 2048
HALF = M_PER // 2
TILE = 512
N_MSG = 2 * (N_DEV - 1)
K = 8192
N_PER = 1024
N_TILES = N_DEV * M_PER // TILE


def _fused_body(x_ref, w_ref, out_ref, top_ref, bot_ref,
                sa_send, sa_recv, sb_send, sb_recv,
                xtile, ostage, w_bf, wstage, ld_sem, st_sem, w_sem):
    my = lax.axis_index("i")
    right = lax.rem(my + 1, N_DEV)
    left = lax.rem(my + N_DEV - 1, N_DEV)

    barrier = pltpu.get_barrier_semaphore()
    for nbr in (left, right):
        pl.semaphore_signal(
            barrier, inc=1, device_id=(nbr,),
            device_id_type=pl.DeviceIdType.MESH,
        )
    pl.semaphore_wait(barrier, 2)

    def msg(ring_a, m, src_own):
        if src_own:
            base = 0 if ring_a else HALF
            src = x_ref.at[pl.ds(base + m * TILE, TILE), :]
        else:
            buf = top_ref if ring_a else bot_ref
            off = jnp.maximum(m - 2, 0) * TILE
            src = buf.at[pl.ds(off, TILE), :]
        dst_buf = top_ref if ring_a else bot_ref
        sems = (sa_send, sa_recv) if ring_a else (sb_send, sb_recv)
        tgt = right if ring_a else left
        return pltpu.make_async_remote_copy(
            src_ref=src,
            dst_ref=dst_buf.at[pl.ds(m * TILE, TILE), :],
            send_sem=sems[0].at[m], recv_sem=sems[1].at[m],
            device_id=(tgt,), device_id_type=pl.DeviceIdType.MESH,
        )

    for m in range(2):
        msg(True, m, src_own=True).start()
        msg(False, m, src_own=True).start()

    W_CH = K // 4
    for c in range(4):
        cw = pltpu.make_async_copy(
            w_ref.at[pl.ds(c * W_CH, W_CH), :], wstage, w_sem
        )
        cw.start()
        cw.wait()
        w_bf[pl.ds(c * W_CH, W_CH), :] = wstage[...].astype(jnp.bfloat16)

    def tile_body(j, carry):
        is_own = j < 4
        ring_a = lax.rem(j, 2) == 0
        m = jnp.maximum((j - 4) // 2, 0)

        @pl.when(jnp.logical_and(jnp.logical_not(is_own), ring_a))
        def _():
            msg(True, m, src_own=False).wait()

            @pl.when(m < N_MSG - 2)
            def _():
                msg(True, m + 2, src_own=False).start()

            pltpu.make_async_copy(
                top_ref.at[pl.ds(m * TILE, TILE), :], xtile, ld_sem
            ).start()

        @pl.when(jnp.logical_and(jnp.logical_not(is_own),
                                 jnp.logical_not(ring_a)))
        def _():
            msg(False, m, src_own=False).wait()

            @pl.when(m < N_MSG - 2)
            def _():
                msg(False, m + 2, src_own=False).start()

            pltpu.make_async_copy(
                bot_ref.at[pl.ds(m * TILE, TILE), :], xtile, ld_sem
            ).start()

        @pl.when(is_own)
        def _():
            pltpu.make_async_copy(
                x_ref.at[pl.ds(j * TILE, TILE), :], xtile, ld_sem
            ).start()

        pltpu.make_async_copy(
            x_ref.at[pl.ds(0, TILE), :], xtile, ld_sem
        ).wait()

        y = jnp.dot(xtile[...], w_bf[...],
                    preferred_element_type=jnp.float32)
        y = y * jax.nn.sigmoid(y)

        @pl.when(j >= 1)
        def _():
            pltpu.make_async_copy(
                ostage, out_ref.at[pl.ds(0, TILE), :], st_sem
            ).wait()

        ostage[...] = y

        sub = lax.rem(m, 2)
        oa = lax.rem(my - 1 - m // 2 + N_DEV, N_DEV)
        ob = lax.rem(my + 1 + m // 2, N_DEV)
        out_row = jnp.where(
            is_own,
            my * M_PER + j * TILE,
            jnp.where(ring_a,
                      oa * M_PER + sub * TILE,
                      ob * M_PER + HALF + sub * TILE),
        )
        pltpu.make_async_copy(
            ostage, out_ref.at[pl.ds(out_row, TILE), :], st_sem
        ).start()
        return carry

    lax.fori_loop(0, N_TILES, tile_body, 0)

    pltpu.make_async_copy(
        ostage, out_ref.at[pl.ds(0, TILE), :], st_sem
    ).wait()


def _fused(x_bf16, w_f32):
    out, _top, _bot = pl.pallas_call(
        _fused_body,
        out_shape=[
            jax.ShapeDtypeStruct((N_DEV * M_PER, N_PER), jnp.float32),
            jax.ShapeDtypeStruct((N_MSG * TILE, K), jnp.bfloat16),
            jax.ShapeDtypeStruct((N_MSG * TILE, K), jnp.bfloat16),
        ],
        in_specs=[
            pl.BlockSpec(memory_space=pl.ANY),
            pl.BlockSpec(memory_space=pl.ANY),
        ],
        out_specs=[
            pl.BlockSpec(memory_space=pl.ANY),
            pl.BlockSpec(memory_space=pl.ANY),
            pl.BlockSpec(memory_space=pl.ANY),
        ],
        scratch_shapes=[
            pltpu.SemaphoreType.DMA((N_MSG,)),
            pltpu.SemaphoreType.DMA((N_MSG,)),
            pltpu.SemaphoreType.DMA((N_MSG,)),
            pltpu.SemaphoreType.DMA((N_MSG,)),
            pltpu.VMEM((TILE, K), jnp.bfloat16),
            pltpu.VMEM((TILE, N_PER), jnp.float32),
            pltpu.VMEM((K, N_PER), jnp.bfloat16),
            pltpu.VMEM((K // 4, N_PER), jnp.float32),
            pltpu.SemaphoreType.DMA,
            pltpu.SemaphoreType.DMA,
            pltpu.SemaphoreType.DMA,
        ],
        compiler_params=pltpu.CompilerParams(
            collective_id=0,
            vmem_limit_bytes=60 * 1024 * 1024,
        ),
    )(x_bf16, w_f32)
    return out


def kernel(x, w_mat):
    x_bf16 = x.astype(jnp.bfloat16)
    return _fused(x_bf16, w_mat)


# device time: 606830 ns/iter; 2.0849x vs baseline; 1.0380x over previous
import jax
import jax.numpy as jnp
from jax import lax
from jax.experimental import pallas as pl
from jax.experimental.pallas import tpu as pltpu

N_DEV = 4
M_PER = 2048
HALF = M_PER // 2
TILE = 512
N_MSG = 2 * (N_DEV - 1)
K = 8192
N_PER = 1024
N_TILES = N_DEV * M_PER // TILE


def _fused_body(x_ref, w_ref, out_ref, top_ref, bot_ref, xbuf_ref,
                sa_send, sa_recv, sb_send, sb_recv,
                xtile, ostage, w_bf, wstage, xf32,
                ld_sem, st_sem, w_sem, xf_sem, xb_sem):
    my = lax.axis_index("i")
    right = lax.rem(my + 1, N_DEV)
    left = lax.rem(my + N_DEV - 1, N_DEV)

    barrier = pltpu.get_barrier_semaphore()
    for nbr in (left, right):
        pl.semaphore_signal(
            barrier, inc=1, device_id=(nbr,),
            device_id_type=pl.DeviceIdType.MESH,
        )
    pl.semaphore_wait(barrier, 2)

    def msg(ring_a, m, src_own):
        if src_own:
            base = 0 if ring_a else HALF
            src = xbuf_ref.at[pl.ds(base + m * TILE, TILE), :]
        else:
            buf = top_ref if ring_a else bot_ref
            off = jnp.maximum(m - 2, 0) * TILE
            src = buf.at[pl.ds(off, TILE), :]
        dst_buf = top_ref if ring_a else bot_ref
        sems = (sa_send, sa_recv) if ring_a else (sb_send, sb_recv)
        tgt = right if ring_a else left
        return pltpu.make_async_remote_copy(
            src_ref=src,
            dst_ref=dst_buf.at[pl.ds(m * TILE, TILE), :],
            send_sem=sems[0].at[m], recv_sem=sems[1].at[m],
            device_id=(tgt,), device_id_type=pl.DeviceIdType.MESH,
        )

    for t in (0, 2, 1, 3):
        ring_a = t < 2
        m = t if ring_a else t - 2
        cf = pltpu.make_async_copy(
            x_ref.at[pl.ds(t * TILE, TILE), :], xf32, xf_sem
        )
        cf.start()
        cf.wait()
        xtile[...] = xf32[...].astype(jnp.bfloat16)
        cb = pltpu.make_async_copy(
            xtile, xbuf_ref.at[pl.ds(t * TILE, TILE), :], xb_sem
        )
        cb.start()
        cb.wait()
        msg(ring_a, m, src_own=True).start()

    W_CH = K // 4
    for c in range(4):
        cw = pltpu.make_async_copy(
            w_ref.at[pl.ds(c * W_CH, W_CH), :], wstage, w_sem
        )
        cw.start()
        cw.wait()
        w_bf[pl.ds(c * W_CH, W_CH), :] = wstage[...].astype(jnp.bfloat16)

    def tile_body(j, carry):
        is_own = j < 4
        ring_a = lax.rem(j, 2) == 0
        m = jnp.maximum((j - 4) // 2, 0)

        @pl.when(jnp.logical_and(jnp.logical_not(is_own), ring_a))
        def _():
            msg(True, m, src_own=False).wait()

            @pl.when(m < N_MSG - 2)
            def _():
                msg(True, m + 2, src_own=False).start()

            pltpu.make_async_copy(
                top_ref.at[pl.ds(m * TILE, TILE), :], xtile, ld_sem
            ).start()

        @pl.when(jnp.logical_and(jnp.logical_not(is_own),
                                 jnp.logical_not(ring_a)))
        def _():
            msg(False, m, src_own=False).wait()

            @pl.when(m < N_MSG - 2)
            def _():
                msg(False, m + 2, src_own=False).start()

            pltpu.make_async_copy(
                bot_ref.at[pl.ds(m * TILE, TILE), :], xtile, ld_sem
            ).start()

        @pl.when(is_own)
        def _():
            pltpu.make_async_copy(
                xbuf_ref.at[pl.ds(j * TILE, TILE), :], xtile, ld_sem
            ).start()

        pltpu.make_async_copy(
            xbuf_ref.at[pl.ds(0, TILE), :], xtile, ld_sem
        ).wait()

        y = jnp.dot(xtile[...], w_bf[...],
                    preferred_element_type=jnp.float32)
        y = y * jax.nn.sigmoid(y)

        @pl.when(j >= 1)
        def _():
            pltpu.make_async_copy(
                ostage, out_ref.at[pl.ds(0, TILE), :], st_sem
            ).wait()

        ostage[...] = y

        sub = lax.rem(m, 2)
        oa = lax.rem(my - 1 - m // 2 + N_DEV, N_DEV)
        ob = lax.rem(my + 1 + m // 2, N_DEV)
        out_row = jnp.where(
            is_own,
            my * M_PER + j * TILE,
            jnp.where(ring_a,
                      oa * M_PER + sub * TILE,
                      ob * M_PER + HALF + sub * TILE),
        )
        pltpu.make_async_copy(
            ostage, out_ref.at[pl.ds(out_row, TILE), :], st_sem
        ).start()
        return carry

    lax.fori_loop(0, N_TILES, tile_body, 0)

    pltpu.make_async_copy(
        ostage, out_ref.at[pl.ds(0, TILE), :], st_sem
    ).wait()


def _fused(x_f32, w_f32):
    out, _top, _bot, _xb = pl.pallas_call(
        _fused_body,
        out_shape=[
            jax.ShapeDtypeStruct((N_DEV * M_PER, N_PER), jnp.float32),
            jax.ShapeDtypeStruct((N_MSG * TILE, K), jnp.bfloat16),
            jax.ShapeDtypeStruct((N_MSG * TILE, K), jnp.bfloat16),
            jax.ShapeDtypeStruct((M_PER, K), jnp.bfloat16),
        ],
        in_specs=[
            pl.BlockSpec(memory_space=pl.ANY),
            pl.BlockSpec(memory_space=pl.ANY),
        ],
        out_specs=[
            pl.BlockSpec(memory_space=pl.ANY),
            pl.BlockSpec(memory_space=pl.ANY),
            pl.BlockSpec(memory_space=pl.ANY),
            pl.BlockSpec(memory_space=pl.ANY),
        ],
        scratch_shapes=[
            pltpu.SemaphoreType.DMA((N_MSG,)),
            pltpu.SemaphoreType.DMA((N_MSG,)),
            pltpu.SemaphoreType.DMA((N_MSG,)),
            pltpu.SemaphoreType.DMA((N_MSG,)),
            pltpu.VMEM((TILE, K), jnp.bfloat16),
            pltpu.VMEM((TILE, N_PER), jnp.float32),
            pltpu.VMEM((K, N_PER), jnp.bfloat16),
            pltpu.VMEM((K // 4, N_PER), jnp.float32),
            pltpu.VMEM((TILE, K), jnp.float32),
            pltpu.SemaphoreType.DMA,
            pltpu.SemaphoreType.DMA,
            pltpu.SemaphoreType.DMA,
            pltpu.SemaphoreType.DMA,
            pltpu.SemaphoreType.DMA,
        ],
        compiler_params=pltpu.CompilerParams(
            collective_id=0,
            vmem_limit_bytes=60 * 1024 * 1024,
        ),
    )(x_f32, w_f32)
    return out


def kernel(x, w_mat):
    return _fused(x, w_mat)
